# baseline (speedup 1.0000x reference)
import jax
import jax.numpy as jnp
from jax import lax
from jax.experimental import pallas as pl
from jax.experimental.pallas import tpu as pltpu, tpu_sc as plsc

B = 16384
D = 16
NC, NS = 2, 16
NW = NC * NS
BPW = B // NW
IDX_CH = 128
NIC = BPW // IDX_CH
NCH = BPW // 16


def _sc_body(uids, iids, wo, wh, wi, out, idx_u, idx_v, urows, vrows, whv,
             outv, sem_u, sem_v):
    wid = lax.axis_index("s") * NC + lax.axis_index("c")
    row0 = wid * NIC

    pltpu.sync_copy(uids.at[pl.ds(row0, NIC)], idx_u)
    pltpu.sync_copy(iids.at[pl.ds(row0, NIC)], idx_v)
    pltpu.sync_copy(wh, whv)

    copies = []
    for j in range(NIC):
        copies.append(pltpu.async_copy(
            wo.at[idx_u.at[j]], urows.at[pl.ds(j * IDX_CH, IDX_CH)], sem_u))
        copies.append(pltpu.async_copy(
            wi.at[idx_v.at[j]], vrows.at[pl.ds(j * IDX_CH, IDX_CH)], sem_v))
    for c in copies:
        c.wait()

    lanes = lax.iota(jnp.int32, 16)
    col_ids = [jnp.full((16,), d, jnp.int32) for d in range(D)]
    wh_rows = [whv[d] for d in range(D)]

    def chunk(c, carry):
        bidx = c * 16 + lanes
        ucols = [plsc.load_gather(urows, [bidx, col_ids[d]]) for d in range(D)]
        vcols = [plsc.load_gather(vrows, [bidx, col_ids[e]]) for e in range(D)]
        acc = jnp.zeros((16,), jnp.float32)
        for d in range(D):
            t = jnp.zeros((16,), jnp.float32)
            for e in range(D):
                t = t + wh_rows[d][e] * vcols[e]
            acc = acc + ucols[d] * t
        outv[pl.ds(c * 16, 16)] = acc
        return carry

    lax.fori_loop(0, NCH, chunk, 0)
    pltpu.sync_copy(outv, out.at[pl.ds(wid * BPW, BPW)])


def kernel(user_ids, item_ids, W_o, W_h, W_i):
    uids = user_ids.reshape(NW * NIC, IDX_CH)
    iids = item_ids.reshape(NW * NIC, IDX_CH)
    mesh = plsc.VectorSubcoreMesh(core_axis_name="c", subcore_axis_name="s")
    f = pl.kernel(
        _sc_body,
        out_type=jax.ShapeDtypeStruct((B,), jnp.float32),
        mesh=mesh,
        compiler_params=pltpu.CompilerParams(
            needs_layout_passes=False, use_tc_tiling_on_sc=False),
        scratch_types=[
            pltpu.VMEM((NIC, IDX_CH), jnp.int32),
            pltpu.VMEM((NIC, IDX_CH), jnp.int32),
            pltpu.VMEM((BPW, D), jnp.float32),
            pltpu.VMEM((BPW, D), jnp.float32),
            pltpu.VMEM((D, D), jnp.float32),
            pltpu.VMEM((BPW,), jnp.float32),
            pltpu.SemaphoreType.DMA,
            pltpu.SemaphoreType.DMA,
        ],
    )
    return f(uids, iids, W_o, W_h, W_i)
